# baseline (speedup 1.0000x reference)
import functools

import jax
import jax.numpy as jnp
from jax import lax
from jax.experimental import pallas as pl
from jax.experimental.pallas import tpu as pltpu
from jax.experimental.pallas import tpu_sc as plsc

_NC, _NS, _L = 2, 16, 16
_NW = _NC * _NS
_CS = 16


def kernel(x, pos_table):
    S, B, D = x.shape
    rows_per_w = S // _NW
    n_chunks = rows_per_w // _CS
    d_vecs = D // _L

    mesh = plsc.VectorSubcoreMesh(core_axis_name="c", subcore_axis_name="s")

    @functools.partial(
        pl.kernel,
        out_type=jax.ShapeDtypeStruct((S, B, D), x.dtype),
        mesh=mesh,
        scratch_types=[
            pltpu.VMEM((_CS, B, D), jnp.float32),
            pltpu.VMEM((_CS, D), jnp.float32),
        ],
    )
    def run(x_hbm, pos_hbm, out_hbm, xv, pv):
        wid = lax.axis_index("s") * _NC + lax.axis_index("c")
        base = wid * rows_per_w

        def chunk(k, carry):
            row0 = base + k * _CS
            pltpu.sync_copy(x_hbm.at[pl.ds(row0, _CS)], xv)
            pltpu.sync_copy(pos_hbm.at[pl.ds(row0, _CS)], pv)
            for sl in range(_CS):
                def jbody(j, c2):
                    off = pl.multiple_of(j * _L, _L)
                    pvec = pv[sl, pl.ds(off, _L)]
                    for b in range(B):
                        xv[sl, b, pl.ds(off, _L)] = (
                            xv[sl, b, pl.ds(off, _L)] + pvec
                        )
                    return c2
                lax.fori_loop(0, d_vecs, jbody, 0)
            pltpu.sync_copy(xv, out_hbm.at[pl.ds(row0, _CS)])
            return carry

        lax.fori_loop(0, n_chunks, chunk, 0)

    return run(x, pos_table[:S])
